# baseline (speedup 1.0000x reference)
import jax
import jax.numpy as jnp
from jax.experimental import pallas as pl
from jax.experimental.pallas import tpu as pltpu

_EPS_BN = 1e-5
_MASKVAL = 1e10
_BIG = 1e30
_BLK = 512


def _fused_kernel(x1_ref, x2_ref, w1t_ref, w2t_ref, bgb1_ref, bgb2_ref,
                  p1_ref, b1f_ref, p2t_ref, b2f_ref,
                  out_ref, feat_s, su_s):
    i = pl.program_id(0)
    n1 = x1_ref.shape[0]
    n2 = p2t_ref.shape[1]

    @pl.when(i == 0)
    def _stats():
        def bn_stats(y, bgb_ref):
            gamma = bgb_ref[1:2, :]
            beta = bgb_ref[2:3, :]
            mu = jnp.mean(y, axis=0, keepdims=True)
            var = jnp.mean((y - mu) * (y - mu), axis=0, keepdims=True)
            s = gamma * jax.lax.rsqrt(var + _EPS_BN)
            return s, beta - mu * s

        y2 = jnp.dot(x2_ref[...], w2t_ref[...],
                     preferred_element_type=jnp.float32) + bgb2_ref[0:1, :]
        s2, u2 = bn_stats(y2, bgb2_ref)
        feat_s[...] = jnp.maximum(y2 * s2 + u2, 0.0)

        y1 = jnp.dot(x1_ref[...], w1t_ref[...],
                     preferred_element_type=jnp.float32) + bgb1_ref[0:1, :]
        s1, u1 = bn_stats(y1, bgb1_ref)
        su_s[0:1, :] = s1
        su_s[1:2, :] = u1

    @pl.when(i > 0)
    def _main():
        r0 = (i - 1) * _BLK
        x1b = x1_ref[pl.ds(r0, _BLK), :]
        y = jnp.dot(x1b, w1t_ref[...],
                    preferred_element_type=jnp.float32) + bgb1_ref[0:1, :]
        h1 = jnp.maximum(y * su_s[0:1, :] + su_s[1:2, :], 0.0)

        p1 = p1_ref[pl.ds(r0, _BLK), :]
        p2t = p2t_ref[...]

        dot = jnp.dot(p1, p2t, preferred_element_type=jnp.float32)
        sq1 = (p1[:, 0:1] * p1[:, 0:1] + p1[:, 1:2] * p1[:, 1:2]) \
            + p1[:, 2:3] * p1[:, 2:3]
        sq2 = (p2t[0:1, :] * p2t[0:1, :] + p2t[1:2, :] * p2t[1:2, :]) \
            + p2t[2:3, :] * p2t[2:3, :]
        d2 = sq1 + sq2 - 2.0 * dot
        same = b1f_ref[pl.ds(r0, _BLK), :] == b2f_ref[...]
        d2m = jnp.where(same, d2, _MASKVAL)

        d2e = None
        for c in range(3):
            diff = p1[:, c:c + 1] - p2t[c:c + 1, :]
            sq = diff * diff
            d2e = sq if d2e is None else d2e + sq

        a = d2m
        msel = []
        sels = []
        for k in range(3):
            m = jnp.min(a, axis=1, keepdims=True)
            sel = a == m
            msel.append(m)
            sels.append(sel)
            if k < 2:
                a = jnp.where(sel, _BIG, a)

        ws = []
        for k in range(3):
            mex = jnp.sum(jnp.where(sels[k], d2e, 0.0), axis=1,
                          keepdims=True)
            w = jnp.where(msel[k] < 1e9,
                          1.0 / (jnp.sqrt(mex) + 1e-8), 0.0)
            ws.append(w)
        inv_norm = 1.0 / (ws[0] + ws[1] + ws[2])
        wmat = jnp.where(sels[0], ws[0] * inv_norm,
                         jnp.where(sels[1], ws[1] * inv_norm,
                                   jnp.where(sels[2], ws[2] * inv_norm,
                                             0.0)))
        nf = jnp.dot(wmat, feat_s[...], preferred_element_type=jnp.float32)
        out_ref[...] = h1 + nf


def kernel(x1, pos1, batch1, x2, pos2, batch2, W1, b1, gamma1, beta1,
           W2, b2, gamma2, beta2):
    n1, c_out = x1.shape
    n2, c_in = x2.shape

    b1f = batch1.astype(jnp.float32)[:, None]
    b2f = batch2.astype(jnp.float32)[None, :]
    p2t = pos2.T

    bgb1 = jnp.stack([b1, gamma1, beta1])
    bgb2 = jnp.stack([b2, gamma2, beta2])

    nblk = n1 // _BLK
    res = lambda i: (0, 0)
    x = pl.pallas_call(
        _fused_kernel,
        grid=(nblk + 1,),
        in_specs=[
            pl.BlockSpec((n1, c_out), res),
            pl.BlockSpec((n2, c_in), res),
            pl.BlockSpec((c_out, c_out), res),
            pl.BlockSpec((c_in, c_out), res),
            pl.BlockSpec((3, c_out), res),
            pl.BlockSpec((3, c_out), res),
            pl.BlockSpec((n1, 3), res),
            pl.BlockSpec((n1, 1), res),
            pl.BlockSpec((3, n2), res),
            pl.BlockSpec((1, n2), res),
        ],
        out_specs=pl.BlockSpec((_BLK, c_out),
                               lambda i: (jnp.maximum(i - 1, 0), 0)),
        out_shape=jax.ShapeDtypeStruct((n1, c_out), jnp.float32),
        scratch_shapes=[
            pltpu.VMEM((n2, c_out), jnp.float32),
            pltpu.VMEM((2, c_out), jnp.float32),
        ],
    )(x1, x2, W1.T, W2.T, bgb1, bgb2, pos1, b1f, p2t, b2f)
    return (x, pos1, batch1)
